# baseline (speedup 1.0000x reference)
import jax
import jax.numpy as jnp
from jax.experimental import pallas as pl
from jax.experimental.pallas import tpu as pltpu


_S = 5
_BM = 40


def _body(x_ref, *rest):
    a_refs = rest[:_S]
    w_ref = rest[_S]
    out_ref = rest[_S + 1]
    h_ref = rest[_S + 2]

    @pl.when(pl.program_id(0) == 0)
    def _():
        h_ref[...] = jnp.dot(
            x_ref[...], w_ref[...], preferred_element_type=jnp.float32
        )

    bm = a_refs[0].shape[0]
    for s, a_ref in enumerate(a_refs):
        out_ref[s * bm:(s + 1) * bm, :] = jnp.dot(
            a_ref[...], h_ref[...], preferred_element_type=jnp.float32
        )


def kernel(x, A, W):
    N, d_in = x.shape
    d_out = W.shape[1]
    bm = _BM

    def _a_spec(s):
        return pl.BlockSpec((bm, N), lambda i, s=s: (_S * i + s, 0))

    return pl.pallas_call(
        _body,
        grid=(N // (_S * bm),),
        in_specs=[
            pl.BlockSpec((N, d_in), lambda i: (0, 0)),
            *[_a_spec(s) for s in range(_S)],
            pl.BlockSpec((d_in, d_out), lambda i: (0, 0)),
        ],
        out_specs=pl.BlockSpec((_S * bm, d_out), lambda i: (i, 0)),
        out_shape=jax.ShapeDtypeStruct((N, d_out), jnp.float32),
        scratch_shapes=[pltpu.VMEM((N, d_out), jnp.float32)],
    )(x, *([A] * _S), W)
